# baseline (speedup 1.0000x reference)
import functools

import jax
import jax.numpy as jnp
from jax import lax
from jax.experimental import pallas as pl
from jax.experimental.pallas import tpu as pltpu
from jax.experimental.pallas import tpu_sc as plsc

_CHUNK = 64
_DPAD = 128
_NHALF = 2
_L = 16


@functools.lru_cache(maxsize=None)
def _build_gather(V, D, B):
    info = plsc.get_sparse_core_info()
    NC, NS = info.num_cores, info.num_subcores
    NW = NC * NS
    assert B % (NW * _NHALF * _CHUNK) == 0, (B, NW)
    b_per_w = B // NW
    b_half = b_per_w // _NHALF
    n_chunks = b_half // _CHUNK
    offs = list(range(0, D - _L + 1, _L))
    if offs[-1] != D - _L:
        offs.append(D - _L)
    mesh = plsc.VectorSubcoreMesh(core_axis_name="c", subcore_axis_name="s")

    @functools.partial(
        pl.kernel,
        mesh=mesh,
        out_type=jax.ShapeDtypeStruct((B, D), jnp.float32),
        scratch_types=[
            pltpu.VMEM_SHARED((V, _DPAD), jnp.float32),
            pltpu.VMEM((b_half,), jnp.int32),
            pltpu.VMEM((_CHUNK, _DPAD), jnp.float32),
            pltpu.VMEM((_CHUNK, _DPAD), jnp.float32),
            pltpu.VMEM((_CHUNK, D), jnp.float32),
            pltpu.VMEM((_CHUNK, D), jnp.float32),
            pltpu.SemaphoreType.DMA,
            pltpu.SemaphoreType.DMA,
            pltpu.SemaphoreType.DMA,
        ],
    )
    def gather_kernel(table_hbm, idx_hbm, out_hbm, tab_s, idx_v,
                      rows0, rows1, pack0, pack1, sg, so0, so1):
        sid = lax.axis_index("s")
        wid = sid * NC + lax.axis_index("c")
        base = wid * b_per_w

        @pl.when(sid == 0)
        def _():
            pltpu.sync_copy(table_hbm, tab_s)

        plsc.subcore_barrier()

        rbufs = (rows0, rows1)
        pbufs = (pack0, pack1)
        osems = (so0, so1)

        def gather(c, b):
            pltpu.async_copy(
                tab_s.at[idx_v.at[pl.ds(c * _CHUNK, _CHUNK)]],
                rbufs[b], sg).wait()

        def repack(b):
            src, dst = rbufs[b], pbufs[b]

            def row(r, carry):
                for o in offs:
                    dst[r, pl.ds(o, _L)] = src[r, pl.ds(o, _L)]
                return carry

            lax.fori_loop(0, _CHUNK, row, 0)

        def start_out(hbase, c, b):
            pltpu.async_copy(
                pbufs[b], out_hbm.at[pl.ds(hbase + c * _CHUNK, _CHUNK)],
                osems[b])

        def wait_out(b):
            pltpu.make_async_copy(
                pbufs[b], out_hbm.at[pl.ds(base, _CHUNK)], osems[b]).wait()

        for h in range(_NHALF):
            hbase = base + h * b_half
            pltpu.sync_copy(idx_hbm.at[pl.ds(hbase, b_half)], idx_v)

            for b in range(2):
                gather(b, b)
                repack(b)
                start_out(hbase, b, b)

            def body(p, carry, hbase=hbase):
                for b in range(2):
                    c = 2 * p + b
                    gather(c, b)
                    wait_out(b)
                    repack(b)
                    start_out(hbase, c, b)
                return carry

            lax.fori_loop(1, n_chunks // 2, body, 0)
            wait_out(0)
            wait_out(1)

    return gather_kernel


def kernel(x, table):
    V, D = table.shape
    B = x.size
    idx = x.reshape(B).astype(jnp.int32)
    table_pad = jnp.pad(table, ((0, 0), (0, _DPAD - D)))
    out = _build_gather(V, D, B)(table_pad, idx)
    return out.reshape(x.shape + (D,))


# speedup vs baseline: 1.0607x; 1.0607x over previous
import functools

import jax
import jax.numpy as jnp
from jax import lax
from jax.experimental import pallas as pl
from jax.experimental.pallas import tpu as pltpu
from jax.experimental.pallas import tpu_sc as plsc

_CHUNK = 64
_DPAD = 128
_NHALF = 2
_BODY = 6
_L = 16


@functools.lru_cache(maxsize=None)
def _build_gather(V, D, B):
    info = plsc.get_sparse_core_info()
    NC, NS = info.num_cores, info.num_subcores
    NW = NC * NS
    assert B % (NW * _NHALF * _CHUNK) == 0, (B, NW)
    b_per_w = B // NW
    b_half = b_per_w // _NHALF
    n_chunks = b_half // _CHUNK
    n_bodies, rem = divmod(n_chunks - 2, _BODY)
    assert rem == 0, n_chunks
    offs = list(range(0, D - _L + 1, _L))
    if offs[-1] != D - _L:
        offs.append(D - _L)
    assert _CHUNK % 8 == 0
    mesh = plsc.VectorSubcoreMesh(core_axis_name="c", subcore_axis_name="s")

    @functools.partial(
        pl.kernel,
        mesh=mesh,
        out_type=jax.ShapeDtypeStruct((B, D), jnp.float32),
        scratch_types=[
            pltpu.VMEM_SHARED((V, _DPAD), jnp.float32),
            pltpu.VMEM((b_half,), jnp.int32),
            pltpu.VMEM((_CHUNK, _DPAD), jnp.float32),
            pltpu.VMEM((_CHUNK, _DPAD), jnp.float32),
            pltpu.VMEM((_CHUNK, D), jnp.float32),
            pltpu.VMEM((_CHUNK, D), jnp.float32),
            pltpu.SemaphoreType.DMA,
            pltpu.SemaphoreType.DMA,
            pltpu.SemaphoreType.DMA,
            pltpu.SemaphoreType.DMA,
        ],
    )
    def gather_kernel(table_hbm, idx_hbm, out_hbm, tab_s, idx_v,
                      rows0, rows1, pack0, pack1, sg0, sg1, so0, so1):
        sid = lax.axis_index("s")
        wid = sid * NC + lax.axis_index("c")
        base = wid * b_per_w

        @pl.when(sid == 0)
        def _():
            pltpu.sync_copy(table_hbm, tab_s)

        plsc.subcore_barrier()

        rbufs = (rows0, rows1)
        pbufs = (pack0, pack1)
        gsems = (sg0, sg1)
        osems = (so0, so1)

        def start_gather(c, b):
            return pltpu.async_copy(
                tab_s.at[idx_v.at[pl.ds(c * _CHUNK, _CHUNK)]],
                rbufs[b], gsems[b])

        def repack(b):
            src, dst = rbufs[b], pbufs[b]

            def rows8(r8, carry):
                r0 = r8 * 8
                for i in range(8):
                    for o in offs:
                        dst[r0 + i, pl.ds(o, _L)] = src[r0 + i, pl.ds(o, _L)]
                return carry

            lax.fori_loop(0, _CHUNK // 8, rows8, 0)

        def start_out(hbase, c, b):
            pltpu.async_copy(
                pbufs[b], out_hbm.at[pl.ds(hbase + c * _CHUNK, _CHUNK)],
                osems[b])

        def wait_out(b):
            pltpu.make_async_copy(
                pbufs[b], out_hbm.at[pl.ds(base, _CHUNK)], osems[b]).wait()

        for h in range(_NHALF):
            hbase = base + h * b_half
            pltpu.sync_copy(idx_hbm.at[pl.ds(hbase, b_half)], idx_v)

            g0 = start_gather(0, 0)
            g1 = start_gather(1, 1)
            g0.wait()
            repack(0)
            start_out(hbase, 0, 0)
            g1.wait()
            repack(1)
            start_out(hbase, 1, 1)

            def body(p, carry, hbase=hbase):
                c0 = 2 + p * _BODY
                g = start_gather(c0, 0)
                for j in range(_BODY):
                    b = j % 2
                    c = c0 + j
                    g_next = (start_gather(c + 1, 1 - b)
                              if j + 1 < _BODY else None)
                    g.wait()
                    wait_out(b)
                    repack(b)
                    start_out(hbase, c, b)
                    g = g_next
                return carry

            lax.fori_loop(0, n_bodies, body, 0)
            wait_out(0)
            wait_out(1)

    return gather_kernel


def kernel(x, table):
    V, D = table.shape
    B = x.size
    idx = x.reshape(B).astype(jnp.int32)
    table_pad = jnp.pad(table, ((0, 0), (0, _DPAD - D)))
    out = _build_gather(V, D, B)(table_pad, idx)
    return out.reshape(x.shape + (D,))


# speedup vs baseline: 1.2860x; 1.2124x over previous
import functools

import jax
import jax.numpy as jnp
from jax import lax
from jax.experimental import pallas as pl
from jax.experimental.pallas import tpu as pltpu
from jax.experimental.pallas import tpu_sc as plsc

_CHUNK = 128
_NHALF = 2


@functools.lru_cache(maxsize=None)
def _build_gather(V, D, B):
    info = plsc.get_sparse_core_info()
    NC, NS = info.num_cores, info.num_subcores
    NW = NC * NS
    assert B % (NW * _NHALF * _CHUNK) == 0, (B, NW)
    b_per_w = B // NW
    b_half = b_per_w // _NHALF
    n_chunks = b_half // _CHUNK
    assert n_chunks % 2 == 0
    mesh = plsc.VectorSubcoreMesh(core_axis_name="c", subcore_axis_name="s")

    @functools.partial(
        pl.kernel,
        mesh=mesh,
        out_type=jax.ShapeDtypeStruct((B, D), jnp.float32),
        scratch_types=[
            pltpu.VMEM_SHARED((V, D), jnp.float32),
            pltpu.VMEM((b_half,), jnp.int32),
            pltpu.VMEM((_CHUNK, D), jnp.float32),
            pltpu.VMEM((_CHUNK, D), jnp.float32),
            pltpu.SemaphoreType.DMA,
            pltpu.SemaphoreType.DMA,
            pltpu.SemaphoreType.DMA,
        ],
    )
    def gather_kernel(table_hbm, idx_hbm, out_hbm, tab_s, idx_v,
                      rows0, rows1, sg, so0, so1):
        sid = lax.axis_index("s")
        wid = sid * NC + lax.axis_index("c")
        base = wid * b_per_w

        @pl.when(sid == 0)
        def _():
            pltpu.sync_copy(table_hbm, tab_s)

        plsc.subcore_barrier()

        bufs = (rows0, rows1)
        osems = (so0, so1)

        def gather(c, b):
            pltpu.async_copy(
                tab_s.at[idx_v.at[pl.ds(c * _CHUNK, _CHUNK)]],
                bufs[b], sg).wait()

        def start_out(hbase, c, b):
            pltpu.async_copy(
                bufs[b], out_hbm.at[pl.ds(hbase + c * _CHUNK, _CHUNK)],
                osems[b])

        def wait_out(b):
            pltpu.make_async_copy(
                bufs[b], out_hbm.at[pl.ds(base, _CHUNK)], osems[b]).wait()

        for h in range(_NHALF):
            hbase = base + h * b_half
            pltpu.sync_copy(idx_hbm.at[pl.ds(hbase, b_half)], idx_v)

            gather(0, 0)
            start_out(hbase, 0, 0)
            gather(1, 1)
            start_out(hbase, 1, 1)

            def body(p, carry, hbase=hbase):
                for b in range(2):
                    c = 2 * p + b
                    wait_out(b)
                    gather(c, b)
                    start_out(hbase, c, b)
                return carry

            lax.fori_loop(1, n_chunks // 2, body, 0)
            wait_out(0)
            wait_out(1)

    return gather_kernel


def kernel(x, table):
    V, D = table.shape
    B = x.size
    idx = x.reshape(B).astype(jnp.int32)
    out = _build_gather(V, D, B)(table, idx)
    return out.reshape(x.shape + (D,))
